# Initial kernel scaffold; baseline (speedup 1.0000x reference)
import jax
import jax.numpy as jnp
from jax.experimental import pallas as pl


def kernel(qagents, state, w_cat, expand, reduce, b2w, b2b):
    raise NotImplementedError("write your pallas kernel here")


# speedup vs baseline: 1.2249x; 1.2249x over previous
import jax
import jax.numpy as jnp
from jax.experimental import pallas as pl
from jax.experimental.pallas import tpu as pltpu


def _qmix_block(q_ref, s_ref, w_ref, bias_ref, b2w_ref, b2b_ref, out_ref):
    f32 = jnp.float32
    s = s_ref[...].astype(jnp.bfloat16)
    proj = jnp.dot(s, w_ref[...], preferred_element_type=f32) + bias_ref[...]
    q = q_ref[...]

    bt = proj.shape[0]
    low = jax.lax.broadcasted_iota(jnp.int32, (bt, 128), 1) < 64

    acc = jnp.zeros((bt, 128), f32)
    for j in range(4):
        w1c = jnp.abs(proj[:, 128 * j:128 * (j + 1)])
        qs = jnp.where(low, q[:, 2 * j:2 * j + 1], q[:, 2 * j + 1:2 * j + 2])
        acc = acc + w1c * qs
    hid = acc + pltpu.roll(acc, 64, axis=1)

    bw = proj[:, 512:640]
    r = pltpu.roll(bw, 64, axis=1)
    b1d = jnp.where(low, bw, r)
    w2d = jnp.abs(jnp.where(low, r, bw))

    mixed = hid + b1d
    mixed = jnp.where(mixed > 0.0, mixed,
                      jnp.exp(jnp.minimum(mixed, 0.0)) - 1.0)

    h2 = jnp.maximum(proj[:, 640:768], 0.0)
    qtot = jnp.sum(mixed * w2d + h2 * b2w_ref[...], axis=1, keepdims=True)
    out_ref[...] = qtot + b2b_ref[...]


def kernel(qagents, state, w_cat, expand, reduce, b2w, b2b):
    del expand, reduce
    f32 = jnp.float32
    B, A = qagents.shape
    S = state.shape[1]
    H = 64
    w0 = A * H

    packed = jnp.concatenate([
        w_cat[:, 0:w0],
        w_cat[:, w0:w0 + H],
        0.5 * w_cat[:, w0 + 128:w0 + 128 + H],
        w_cat[:, w0 + 256:w0 + 256 + S],
    ], axis=1)
    w_bf = packed[:S, :].astype(jnp.bfloat16)
    bias = packed[S:S + 1, :]

    BB = 1024
    grid_b = pl.cdiv(B, BB)
    b_pad = grid_b * BB
    if b_pad != B:
        qagents = jnp.pad(qagents, ((0, b_pad - B), (0, 0)))
        state = jnp.pad(state, ((0, b_pad - B), (0, 0)))

    out = pl.pallas_call(
        _qmix_block,
        out_shape=jax.ShapeDtypeStruct((b_pad, 1), f32),
        grid=(grid_b,),
        in_specs=[
            pl.BlockSpec((BB, A), lambda i: (i, 0)),
            pl.BlockSpec((BB, S), lambda i: (i, 0)),
            pl.BlockSpec((S, 768), lambda i: (0, 0)),
            pl.BlockSpec((1, 768), lambda i: (0, 0)),
            pl.BlockSpec((1, 128), lambda i: (0, 0)),
            pl.BlockSpec((1, 1), lambda i: (0, 0)),
        ],
        out_specs=pl.BlockSpec((BB, 1), lambda i: (i, 0)),
        compiler_params=pltpu.CompilerParams(
            dimension_semantics=("parallel",)),
    )(qagents, state, w_bf, bias, b2w, b2b)
    return out.reshape(-1)[:B]


# speedup vs baseline: 1.3368x; 1.0914x over previous
import jax
import jax.numpy as jnp
from jax.experimental import pallas as pl
from jax.experimental.pallas import tpu as pltpu


def _qmix_block(q_ref, s_ref, w_ref, bias_ref, b2w_ref, b2b_ref, out_ref):
    f32 = jnp.float32
    s = s_ref[...].astype(jnp.bfloat16)
    proj = jnp.dot(s, w_ref[...], preferred_element_type=f32) + bias_ref[...]
    q = q_ref[...]

    bt = proj.shape[0]
    low = jax.lax.broadcasted_iota(jnp.int32, (bt, 128), 1) < 64

    acc = jnp.zeros((bt, 128), f32)
    for j in range(4):
        w1c = jnp.abs(proj[:, 128 * j:128 * (j + 1)])
        qs = jnp.where(low, q[:, 2 * j:2 * j + 1], q[:, 2 * j + 1:2 * j + 2])
        acc = acc + w1c * qs
    hid = acc + pltpu.roll(acc, 64, axis=1)

    bw = proj[:, 512:640]
    r = pltpu.roll(bw, 64, axis=1)
    b1d = jnp.where(low, bw, r)
    w2d = jnp.abs(jnp.where(low, r, bw))

    mixed = hid + b1d
    mixed = jnp.where(mixed > 0.0, mixed,
                      jnp.exp(jnp.minimum(mixed, 0.0)) - 1.0)

    h2 = jnp.maximum(proj[:, 640:768], 0.0)
    qtot = jnp.sum(mixed * w2d + h2 * b2w_ref[...], axis=1, keepdims=True)
    out_ref[...] = qtot + b2b_ref[...]


def kernel(qagents, state, w_cat, expand, reduce, b2w, b2b):
    del expand, reduce
    f32 = jnp.float32
    B, A = qagents.shape
    S = state.shape[1]
    H = 64
    w0 = A * H

    packed = jnp.concatenate([
        w_cat[:, 0:w0],
        w_cat[:, w0:w0 + H],
        0.5 * w_cat[:, w0 + 128:w0 + 128 + H],
        w_cat[:, w0 + 256:w0 + 256 + S],
    ], axis=1)
    w_bf = packed[:S, :].astype(jnp.bfloat16)
    bias = packed[S:S + 1, :]

    BB = 4096
    grid_b = pl.cdiv(B, BB)
    b_pad = grid_b * BB
    if b_pad != B:
        qagents = jnp.pad(qagents, ((0, b_pad - B), (0, 0)))
        state = jnp.pad(state, ((0, b_pad - B), (0, 0)))

    out = pl.pallas_call(
        _qmix_block,
        out_shape=jax.ShapeDtypeStruct((b_pad, 1), f32),
        grid=(grid_b,),
        in_specs=[
            pl.BlockSpec((BB, A), lambda i: (i, 0)),
            pl.BlockSpec((BB, S), lambda i: (i, 0)),
            pl.BlockSpec((S, 768), lambda i: (0, 0)),
            pl.BlockSpec((1, 768), lambda i: (0, 0)),
            pl.BlockSpec((1, 128), lambda i: (0, 0)),
            pl.BlockSpec((1, 1), lambda i: (0, 0)),
        ],
        out_specs=pl.BlockSpec((BB, 1), lambda i: (i, 0)),
        compiler_params=pltpu.CompilerParams(
            dimension_semantics=("parallel",)),
    )(qagents, state, w_bf, bias, b2w, b2b)
    return out.reshape(-1)[:B]
